# Initial kernel scaffold; baseline (speedup 1.0000x reference)
import jax
import jax.numpy as jnp
from jax.experimental import pallas as pl


def kernel(x, edge_index, Wl1, bl1, Wr1, br1, att1, bias1, Wl2, bl2, Wr2, br2, att2, bias2):
    raise NotImplementedError("write your pallas kernel here")


# speedup vs baseline: 19.0122x; 19.0122x over previous
import functools

import jax
import jax.numpy as jnp
from jax import lax
from jax.experimental import pallas as pl
from jax.experimental.pallas import tpu as pltpu
from jax.experimental.pallas import tpu_sc as plsc

N = 10000
E = 320000
F = 128
HEADS = 8
EXTW = 144

NC = 2
NS = 16
NW = NC * NS
BATCH = 128
PER_W = 10112
E_PAD = NW * PER_W
NB = PER_W // BATCH

@functools.cache
def _sc_mesh():
    return plsc.VectorSubcoreMesh(core_axis_name="c", subcore_axis_name="s")


def _sc_gather(xl, xr, srcp, dstp):
    @functools.partial(
        pl.kernel, mesh=_sc_mesh(),
        out_type=[jax.ShapeDtypeStruct((E_PAD, F), jnp.float32),
                  jax.ShapeDtypeStruct((E_PAD, F), jnp.float32)],
        scratch_types=[pltpu.VMEM((BATCH,), jnp.int32),
                       pltpu.VMEM((BATCH,), jnp.int32),
                       pltpu.VMEM((BATCH, F), jnp.float32),
                       pltpu.VMEM((BATCH, F), jnp.float32),
                       pltpu.SemaphoreType.DMA,
                       pltpu.SemaphoreType.DMA],
    )
    def k(xl_hbm, xr_hbm, src_hbm, dst_hbm, xlo_hbm, xro_hbm,
          si, di, rl, rr, s1, s2):
        wid = lax.axis_index("s") * NC + lax.axis_index("c")
        base0 = wid * PER_W

        def body(b, carry):
            base = base0 + b * BATCH
            pltpu.sync_copy(src_hbm.at[pl.ds(base, BATCH)], si)
            pltpu.sync_copy(dst_hbm.at[pl.ds(base, BATCH)], di)
            cl = pltpu.async_copy(xl_hbm.at[si], rl, s1)
            cr = pltpu.async_copy(xr_hbm.at[di], rr, s2)
            cl.wait()
            cr.wait()
            pltpu.sync_copy(rl, xlo_hbm.at[pl.ds(base, BATCH)])
            pltpu.sync_copy(rr, xro_hbm.at[pl.ds(base, BATCH)])
            return carry

        lax.fori_loop(0, NB, body, 0)

    return k(xl, xr, srcp, dstp)


def _sc_scatter(ext_msg, ext_wb, dstp, zinit):
    PER_S = E_PAD // NS
    NBS = PER_S // BATCH

    @functools.partial(
        pl.kernel, mesh=_sc_mesh(),
        out_type=jax.ShapeDtypeStruct((2 * N, F), jnp.float32),
        scratch_types=[pltpu.VMEM((BATCH,), jnp.int32),
                       pltpu.VMEM((BATCH, F), jnp.float32),
                       pltpu.VMEM_SHARED((N, F), jnp.float32)],
    )
    def k(msg_hbm, wb_hbm, dst_hbm, z_hbm, acc_hbm, di, rv, shared):
        c = lax.axis_index("c")
        s = lax.axis_index("s")
        rows = 1000

        @pl.when(s < 10)
        def _():
            pltpu.sync_copy(z_hbm.at[pl.ds(s * rows, rows)],
                            shared.at[pl.ds(s * rows, rows)])
        plsc.subcore_barrier()

        def body_of(src_hbm):
            def body(b, carry):
                base = s * PER_S + b * BATCH
                pltpu.sync_copy(dst_hbm.at[pl.ds(base, BATCH)], di)
                pltpu.sync_copy(src_hbm.at[pl.ds(base, BATCH)], rv)
                pltpu.sync_copy(rv, shared.at[di], add=True)
                return carry
            return body

        @pl.when(c == 0)
        def _():
            lax.fori_loop(0, NBS, body_of(msg_hbm), 0)

        @pl.when(c == 1)
        def _():
            lax.fori_loop(0, NBS, body_of(wb_hbm), 0)

        plsc.subcore_barrier()

        @pl.when(s < 10)
        def _():
            pltpu.sync_copy(shared.at[pl.ds(s * rows, rows)],
                            acc_hbm.at[pl.ds(c * N + s * rows, rows)])

    return k(ext_msg, ext_wb, dstp, zinit)


def _mm2(x, Wl, bl, Wr, br):
    BN = 1000

    def body(x_ref, wl_ref, bl_ref, wr_ref, br_ref, xl_ref, xr_ref):
        xb = x_ref[...]
        xl_ref[...] = jnp.dot(xb, wl_ref[...],
                              preferred_element_type=jnp.float32) + bl_ref[...]
        xr_ref[...] = jnp.dot(xb, wr_ref[...],
                              preferred_element_type=jnp.float32) + br_ref[...]

    full = pl.BlockSpec((F, F), lambda i: (0, 0))
    bias = pl.BlockSpec((1, F), lambda i: (0, 0))
    return pl.pallas_call(
        body,
        grid=(N // BN,),
        in_specs=[pl.BlockSpec((BN, F), lambda i: (i, 0)), full, bias, full, bias],
        out_specs=[pl.BlockSpec((BN, F), lambda i: (i, 0)),
                   pl.BlockSpec((BN, F), lambda i: (i, 0))],
        out_shape=[jax.ShapeDtypeStruct((N, F), jnp.float32),
                   jax.ShapeDtypeStruct((N, F), jnp.float32)],
    )(x, Wl, bl, Wr, br)


def _edge(XL, XR, attrow, S, ST):
    BE = 4096

    def body(xl_ref, xr_ref, att_ref, s_ref, st_ref, msg_ref, wb_ref):
        i = pl.program_id(0)
        a = xl_ref[...]
        s = a + xr_ref[...]
        lr = jnp.where(s > 0, s, 0.2 * s)
        p = lr * att_ref[...]
        alpha = jnp.dot(p, s_ref[...], preferred_element_type=jnp.float32)
        rows = lax.broadcasted_iota(jnp.int32, (BE, 1), 0) + i * BE
        w = jnp.where(rows < E, jnp.exp(alpha), 0.0)
        wb = jnp.dot(w, st_ref[...], preferred_element_type=jnp.float32)
        msg_ref[...] = a * wb
        wb_ref[...] = wb

    return pl.pallas_call(
        body,
        grid=(E_PAD // BE,),
        in_specs=[pl.BlockSpec((BE, F), lambda i: (i, 0)),
                  pl.BlockSpec((BE, F), lambda i: (i, 0)),
                  pl.BlockSpec((1, F), lambda i: (0, 0)),
                  pl.BlockSpec((F, HEADS), lambda i: (0, 0)),
                  pl.BlockSpec((HEADS, F), lambda i: (0, 0))],
        out_specs=[pl.BlockSpec((BE, F), lambda i: (i, 0)),
                   pl.BlockSpec((BE, F), lambda i: (i, 0))],
        out_shape=[jax.ShapeDtypeStruct((E_PAD, F), jnp.float32),
                   jax.ShapeDtypeStruct((E_PAD, F), jnp.float32)],
    )(XL, XR, attrow, S, ST)


def _comb_mm(num, den, Wl, bl, Wr, br):
    BN = 1000

    def body(n_ref, d_ref, wl_ref, bl_ref, wr_ref, br_ref, xl_ref, xr_ref):
        h = n_ref[...] / (d_ref[...] + 1e-16)
        xl_ref[...] = jnp.dot(h, wl_ref[...],
                              preferred_element_type=jnp.float32) + bl_ref[...]
        xr_ref[...] = jnp.dot(h, wr_ref[...],
                              preferred_element_type=jnp.float32) + br_ref[...]

    full = pl.BlockSpec((F, F), lambda i: (0, 0))
    bias = pl.BlockSpec((1, F), lambda i: (0, 0))
    return pl.pallas_call(
        body,
        grid=(N // BN,),
        in_specs=[pl.BlockSpec((BN, F), lambda i: (i, 0)),
                  pl.BlockSpec((BN, F), lambda i: (i, 0)),
                  full, bias, full, bias],
        out_specs=[pl.BlockSpec((BN, F), lambda i: (i, 0)),
                   pl.BlockSpec((BN, F), lambda i: (i, 0))],
        out_shape=[jax.ShapeDtypeStruct((N, F), jnp.float32),
                   jax.ShapeDtypeStruct((N, F), jnp.float32)],
    )(num, den, Wl, bl, Wr, br)


def _finalize(num, den, bias):
    BN = 1000

    def body(n_ref, d_ref, b_ref, z_ref):
        h = n_ref[...] / (d_ref[...] + 1e-16) + b_ref[...]
        nrm = jnp.sqrt(jnp.sum(h * h, axis=1, keepdims=True))
        z_ref[...] = h / jnp.maximum(nrm, 1e-12)

    return pl.pallas_call(
        body,
        grid=(N // BN,),
        in_specs=[pl.BlockSpec((BN, F), lambda i: (i, 0)),
                  pl.BlockSpec((BN, F), lambda i: (i, 0)),
                  pl.BlockSpec((1, F), lambda i: (0, 0))],
        out_specs=pl.BlockSpec((BN, F), lambda i: (i, 0)),
        out_shape=jax.ShapeDtypeStruct((N, F), jnp.float32),
    )(num, den, bias)


def _decode(z):
    BR = 200

    def body(zx_ref, zy_ref, out_ref):
        zz = lax.dot_general(zx_ref[...], zy_ref[...],
                             (((1,), (1,)), ((), ())),
                             preferred_element_type=jnp.float32)
        out_ref[...] = jax.nn.sigmoid(zz)

    return pl.pallas_call(
        body,
        grid=(N // BR,),
        in_specs=[pl.BlockSpec((BR, F), lambda i: (i, 0)),
                  pl.BlockSpec((N, F), lambda i: (0, 0))],
        out_specs=pl.BlockSpec((BR, N), lambda i: (i, 0)),
        out_shape=jax.ShapeDtypeStruct((N, N), jnp.float32),
    )(z, z)


def kernel(x, edge_index, Wl1, bl1, Wr1, br1, att1, bias1,
           Wl2, bl2, Wr2, br2, att2, bias2):
    src = edge_index[0]
    dst = edge_index[1]
    pad = jnp.zeros((E_PAD - E,), jnp.int32)
    srcp = jnp.concatenate([src, pad])
    dstp = jnp.concatenate([dst, pad])

    S1 = jnp.kron(jnp.eye(HEADS, dtype=jnp.float32),
                  jnp.ones((F // HEADS, 1), jnp.float32))
    ST1 = S1.T
    S2 = jnp.ones((F, HEADS), jnp.float32)
    ST2 = jnp.ones((HEADS, F), jnp.float32) / HEADS
    zinit = jnp.zeros((N, F), jnp.float32)

    xl1, xr1 = _mm2(x, Wl1, bl1.reshape(1, F), Wr1, br1.reshape(1, F))
    XL1, XR1 = _sc_gather(xl1, xr1, srcp, dstp)
    msg1, wb1 = _edge(XL1, XR1, att1.reshape(1, F), S1, ST1)
    acc1 = _sc_scatter(msg1, wb1, dstp, zinit)
    xl2, xr2 = _comb_mm(acc1[:N], acc1[N:], Wl2, bl2.reshape(1, F),
                        Wr2, br2.reshape(1, F))
    XL2, XR2 = _sc_gather(xl2, xr2, srcp, dstp)
    msg2, wb2 = _edge(XL2, XR2, att2.reshape(1, F), S2, ST2)
    acc2 = _sc_scatter(msg2, wb2, dstp, zinit)
    z = _finalize(acc2[:N], acc2[N:], bias2.reshape(1, F))
    A = _decode(z)
    return (A, z)
